# Initial kernel scaffold; baseline (speedup 1.0000x reference)
import jax
import jax.numpy as jnp
from jax.experimental import pallas as pl


def kernel(eb_input, eb_offset, W0, W1, W2):
    raise NotImplementedError("write your pallas kernel here")


# speedup vs baseline: 13.1639x; 13.1639x over previous
import functools

import jax
import jax.numpy as jnp
from jax import lax
from jax.experimental import pallas as pl
from jax.experimental.pallas import tpu as pltpu
from jax.experimental.pallas import tpu_sc as plsc

N_INDICES = 819200
EMBED_DIM = 3
NUM_CORES = 2
NUM_SUBCORES = 16
NW = NUM_CORES * NUM_SUBCORES
B = N_INDICES // NW
L = 16
SCALES = (5.0, 10.0, 6.0)

_mesh = plsc.VectorSubcoreMesh(
    core_axis_name="c", subcore_axis_name="s",
    num_cores=NUM_CORES, num_subcores=NUM_SUBCORES)


@functools.partial(
    pl.kernel,
    out_type=jax.ShapeDtypeStruct((NW, 9, L), jnp.float32),
    mesh=_mesh,
    scratch_types=[
        pltpu.VMEM((B,), jnp.int32),
        pltpu.VMEM((B,), jnp.int32),
        pltpu.VMEM((B,), jnp.int32),
        pltpu.VMEM((B,), jnp.int32),
        pltpu.VMEM((B,), jnp.float32),
        pltpu.VMEM((9, L), jnp.float32),
        pltpu.SemaphoreType.DMA,
    ],
)
def _eb_sum_kernel(idx_hbm, w0_hbm, w1_hbm, w2_hbm, out_hbm,
                   idx_v, i0_v, i1_v, i2_v, col_v, out_v, sem):
    wid = lax.axis_index("s") * NUM_CORES + lax.axis_index("c")
    base = wid * B
    pltpu.sync_copy(idx_hbm.at[pl.ds(base, B)], idx_v)

    def build(i, _, idx_v=idx_v, i0_v=i0_v, i1_v=i1_v, i2_v=i2_v):
        t3 = idx_v[pl.ds(i * L, L)] * 3
        i0_v[pl.ds(i * L, L)] = t3
        i1_v[pl.ds(i * L, L)] = t3 + 1
        i2_v[pl.ds(i * L, L)] = t3 + 2
        return 0

    lax.fori_loop(0, B // L, build, 0)

    for g, (w_hbm, scale) in enumerate(
            ((w0_hbm, SCALES[0]), (w1_hbm, SCALES[1]), (w2_hbm, SCALES[2]))):
        for c, ic_v in enumerate((i0_v, i1_v, i2_v)):
            pltpu.async_copy(w_hbm.at[ic_v], col_v, sem).wait()

            def body(i, acc, col_v=col_v):
                return acc + col_v[pl.ds(i * L, L)]

            acc = lax.fori_loop(0, B // L, body, jnp.zeros((L,), jnp.float32))
            out_v[g * 3 + c, :] = acc * scale

    pltpu.sync_copy(out_v, out_hbm.at[wid])


def kernel(eb_input, eb_offset, W0, W1, W2):
    del eb_offset
    idx = eb_input.astype(jnp.int32)
    partials = _eb_sum_kernel(
        idx, W0.reshape(-1), W1.reshape(-1), W2.reshape(-1))
    return jnp.sum(partials, axis=(0, 2)).reshape(3, EMBED_DIM)


# speedup vs baseline: 13.2410x; 1.0059x over previous
import functools

import jax
import jax.numpy as jnp
from jax import lax
from jax.experimental import pallas as pl
from jax.experimental.pallas import tpu as pltpu
from jax.experimental.pallas import tpu_sc as plsc

N_INDICES = 819200
EMBED_DIM = 3
NUM_CORES = 2
NUM_SUBCORES = 16
NW = NUM_CORES * NUM_SUBCORES
B = N_INDICES // NW
L = 16
K = 8
CH = B // K
SCALES = (5.0, 10.0, 6.0)

_mesh = plsc.VectorSubcoreMesh(
    core_axis_name="c", subcore_axis_name="s",
    num_cores=NUM_CORES, num_subcores=NUM_SUBCORES)


@functools.partial(
    pl.kernel,
    out_type=jax.ShapeDtypeStruct((NW, 9, L), jnp.float32),
    mesh=_mesh,
    scratch_types=[
        pltpu.VMEM((B,), jnp.int32),
        pltpu.VMEM((B,), jnp.int32),
        pltpu.VMEM((B,), jnp.int32),
        pltpu.VMEM((B,), jnp.float32),
        pltpu.VMEM((B,), jnp.float32),
        pltpu.VMEM((9, L), jnp.float32),
        pltpu.SemaphoreType.DMA,
        pltpu.SemaphoreType.DMA,
    ],
)
def _eb_sum_kernel(idx_hbm, w0_hbm, w1_hbm, w2_hbm, out_hbm,
                   i0_v, i1_v, i2_v, col_a, col_b, out_v, sem_a, sem_b):
    wid = lax.axis_index("s") * NUM_CORES + lax.axis_index("c")
    base = wid * B
    pltpu.sync_copy(idx_hbm.at[pl.ds(base, B)], i0_v)

    def build(i, _, i0_v=i0_v, i1_v=i1_v, i2_v=i2_v):
        t3 = i0_v[pl.ds(i * L, L)] * 3
        i0_v[pl.ds(i * L, L)] = t3
        i1_v[pl.ds(i * L, L)] = t3 + 1
        i2_v[pl.ds(i * L, L)] = t3 + 2
        return 0

    lax.fori_loop(0, B // L, build, 0)

    idx_refs = (i0_v, i1_v, i2_v)
    pairs = [(w, c, s)
             for w, s in ((w0_hbm, SCALES[0]), (w1_hbm, SCALES[1]),
                          (w2_hbm, SCALES[2]))
             for c in range(3)]
    bufs = (col_a, col_b)
    sems = (sem_a, sem_b)

    def fire(p):
        w_hbm, c, _ = pairs[p]
        buf, sem, ic_v = bufs[p % 2], sems[p % 2], idx_refs[c]
        return [
            pltpu.async_copy(w_hbm.at[ic_v.at[pl.ds(k * CH, CH)]],
                             buf.at[pl.ds(k * CH, CH)], sem)
            for k in range(K)
        ]

    descs = fire(0)
    for p in range(9):
        for d in descs:
            d.wait()
        if p + 1 < 9:
            descs = fire(p + 1)
        buf, (_, c, scale) = bufs[p % 2], pairs[p]

        def body(i, carry, buf=buf):
            a0, a1, a2, a3 = carry
            e = i * (4 * L)
            a0 = a0 + buf[pl.ds(e, L)]
            a1 = a1 + buf[pl.ds(e + L, L)]
            a2 = a2 + buf[pl.ds(e + 2 * L, L)]
            a3 = a3 + buf[pl.ds(e + 3 * L, L)]
            return a0, a1, a2, a3

        zero = jnp.zeros((L,), jnp.float32)
        a0, a1, a2, a3 = lax.fori_loop(0, B // (4 * L), body,
                                       (zero, zero, zero, zero))
        out_v[(p // 3) * 3 + c, :] = (a0 + a1 + a2 + a3) * scale

    pltpu.sync_copy(out_v, out_hbm.at[wid])


def kernel(eb_input, eb_offset, W0, W1, W2):
    del eb_offset
    idx = eb_input.astype(jnp.int32)
    partials = _eb_sum_kernel(
        idx, W0.reshape(-1), W1.reshape(-1), W2.reshape(-1))
    return jnp.sum(partials, axis=(0, 2)).reshape(3, EMBED_DIM)


# speedup vs baseline: 109.8423x; 8.2956x over previous
import functools

import jax
import jax.numpy as jnp
from jax import lax
from jax.experimental import pallas as pl
from jax.experimental.pallas import tpu as pltpu
from jax.experimental.pallas import tpu_sc as plsc

NUM_EMBEDDINGS = 1_000_000
N_INDICES = 819200
EMBED_DIM = 3
NUM_CORES = 2
NUM_SUBCORES = 16
L = 16
NW = NUM_CORES * NUM_SUBCORES
BS = N_INDICES // NUM_SUBCORES
IC = 2048
NIC = BS // IC
SCALES = (5.0, 10.0, 6.0)

HALF = NUM_EMBEDDINGS // NUM_CORES
DUMMY = HALF
CNT_WORDS = HALF + 8

VC = 4000
NCH = HALF // VC
CPT = NCH // NUM_SUBCORES
EXTRA = NCH - CPT * NUM_SUBCORES

_mesh = plsc.VectorSubcoreMesh(
    core_axis_name="c", subcore_axis_name="s",
    num_cores=NUM_CORES, num_subcores=NUM_SUBCORES)


@functools.partial(
    pl.kernel,
    out_type=jax.ShapeDtypeStruct((NW, 9, L), jnp.float32),
    mesh=_mesh,
    scratch_types=[
        pltpu.VMEM((IC // 128, 128), jnp.int32),
        pltpu.VMEM((128,), jnp.float32),
        pltpu.VMEM((VC,), jnp.float32),
        pltpu.VMEM((VC,), jnp.float32),
        pltpu.VMEM((VC,), jnp.float32),
        pltpu.VMEM((VC,), jnp.float32),
        pltpu.VMEM((VC,), jnp.float32),
        pltpu.VMEM((VC,), jnp.float32),
        pltpu.VMEM((VC,), jnp.float32),
        pltpu.VMEM((9, L), jnp.float32),
        pltpu.VMEM_SHARED((CNT_WORDS,), jnp.float32),
        pltpu.SemaphoreType.DMA,
        pltpu.SemaphoreType.DMA,
    ],
)
def _eb_hist_kernel(idx_hbm, w00, w01, w02, w10, w11, w12, w20, w21, w22,
                    out_hbm,
                    idx_v, ones_v, cnt_v, wa0, wa1, wa2, wb0, wb1, wb2,
                    out_v, counts_sp, sem_a, sem_b):
    cid = lax.axis_index("c")
    sid = lax.axis_index("s")
    wid = sid * NUM_CORES + cid

    zero16 = jnp.zeros((L,), jnp.float32)
    one16 = zero16 + 1.0

    def fill_ones(i, _, ones_v=ones_v):
        ones_v[pl.ds(i * L, L)] = one16
        return 0

    lax.fori_loop(0, 128 // L, fill_ones, 0)

    def fill_zero(i, _, cnt_v=cnt_v):
        cnt_v[pl.ds(i * L, L)] = zero16
        return 0

    lax.fori_loop(0, VC // L, fill_zero, 0)

    def local_start(it, sid=sid):
        return pl.multiple_of((sid + NUM_SUBCORES * it) * VC, 8)

    n_my_chunks = jnp.where(sid < EXTRA, CPT + 1, CPT)

    def zero_chunk(it, _, cnt_v=cnt_v, counts_sp=counts_sp):
        pltpu.sync_copy(cnt_v, counts_sp.at[pl.ds(local_start(it), VC)])
        return 0

    lax.fori_loop(0, n_my_chunks, zero_chunk, 0)
    plsc.subcore_barrier()

    lo = cid * HALF

    def hist_chunk(j, _, idx_v=idx_v, ones_v=ones_v, counts_sp=counts_sp):
        pltpu.sync_copy(idx_hbm.at[sid, j], idx_v)

        for r in range(IC // 128):
            def remap(i, _, idx_v=idx_v, r=r):
                s = pl.ds(i * L, L)
                rel = idx_v[r, s] - lo
                ok = (rel >= 0) & (rel < HALF)
                idx_v[r, s] = jnp.where(ok, rel, DUMMY)
                return 0

            lax.fori_loop(0, 128 // L, remap, 0)
        for r in range(IC // 128):
            pltpu.sync_copy(ones_v, counts_sp.at[idx_v.at[r]], add=True)
        return 0

    for turn in range(NUM_SUBCORES):
        @pl.when(sid == turn)
        def _turn():
            lax.fori_loop(0, NIC, hist_chunk, 0)
        plsc.subcore_barrier()

    tables = (((w00, w01, w02), SCALES[0]),
              ((w10, w11, w12), SCALES[1]),
              ((w20, w21, w22), SCALES[2]))
    slots = ((wa0, wa1, wa2), (wb0, wb1, wb2))
    sems = (sem_a, sem_b)

    def fire_table(g, it, slot, lo=lo):
        planes = tables[g][0]
        start = lo + local_start(it)
        return [pltpu.async_copy(planes[c].at[pl.ds(start, VC)],
                                 slots[slot][c], sems[slot])
                for c in range(EMBED_DIM)]

    def reduce_table(accs, g, slot, cnt_v=cnt_v):
        w0_v, w1_v, w2_v = slots[slot]

        def body(i, carry, w0_v=w0_v, w1_v=w1_v, w2_v=w2_v):
            a0, a1, a2 = carry
            for u in range(2):
                s = pl.ds((2 * i + u) * L, L)
                cnt16 = cnt_v[s]
                a0 = a0 + cnt16 * w0_v[s]
                a1 = a1 + cnt16 * w1_v[s]
                a2 = a2 + cnt16 * w2_v[s]
            return a0, a1, a2

        new = lax.fori_loop(0, VC // (2 * L), body,
                            (accs[3 * g], accs[3 * g + 1], accs[3 * g + 2]))
        accs = list(accs)
        accs[3 * g], accs[3 * g + 1], accs[3 * g + 2] = new
        return tuple(accs)

    def chunk_body(it, accs):
        pltpu.sync_copy(counts_sp.at[pl.ds(local_start(it), VC)], cnt_v)
        descs = fire_table(0, it, 0)
        for g in range(3):
            nxt = fire_table(g + 1, it, (g + 1) % 2) if g < 2 else None
            for d in descs:
                d.wait()
            accs = reduce_table(accs, g, g % 2)
            descs = nxt
        return accs

    accs = lax.fori_loop(0, n_my_chunks, chunk_body,
                         tuple(zero16 for _ in range(9)))

    for p in range(9):
        out_v[p, :] = accs[p] * tables[p // 3][1]

    pltpu.sync_copy(out_v, out_hbm.at[wid])


def kernel(eb_input, eb_offset, W0, W1, W2):
    del eb_offset
    idx = eb_input.astype(jnp.int32).reshape(NUM_SUBCORES, NIC, IC // 128, 128)
    planes = [W[:, c] for W in (W0, W1, W2) for c in range(EMBED_DIM)]
    partials = _eb_hist_kernel(idx, *planes)
    return jnp.sum(partials, axis=(0, 2)).reshape(3, EMBED_DIM)


# speedup vs baseline: 167.0891x; 1.5212x over previous
import functools

import jax
import jax.numpy as jnp
from jax import lax
from jax.experimental import pallas as pl
from jax.experimental.pallas import tpu as pltpu
from jax.experimental.pallas import tpu_sc as plsc

NUM_EMBEDDINGS = 1_000_000
N_INDICES = 819200
EMBED_DIM = 3
NUM_CORES = 2
NUM_SUBCORES = 16
L = 16
NW = NUM_CORES * NUM_SUBCORES
BS = N_INDICES // NUM_SUBCORES
IC = 6400
NIC = BS // IC
SCALES = (5.0, 10.0, 6.0)

HALF = NUM_EMBEDDINGS // NUM_CORES
DUMMY = HALF
CNT_WORDS = HALF + 8

VC = 800
NCH = HALF // VC
CPT = NCH // NUM_SUBCORES
EXTRA = NCH - CPT * NUM_SUBCORES

_mesh = plsc.VectorSubcoreMesh(
    core_axis_name="c", subcore_axis_name="s",
    num_cores=NUM_CORES, num_subcores=NUM_SUBCORES)


@functools.partial(
    pl.kernel,
    out_type=jax.ShapeDtypeStruct((NW, 9, L), jnp.float32),
    mesh=_mesh,
    scratch_types=[
        [pltpu.VMEM((IC,), jnp.int32) for _ in range(NIC)],
        pltpu.VMEM((IC,), jnp.float32),
        pltpu.VMEM((VC,), jnp.float32),
        pltpu.VMEM((VC,), jnp.float32),
        pltpu.VMEM((VC,), jnp.float32),
        pltpu.VMEM((VC,), jnp.float32),
        pltpu.VMEM((VC,), jnp.float32),
        pltpu.VMEM((VC,), jnp.float32),
        pltpu.VMEM((VC,), jnp.float32),
        pltpu.VMEM((9, L), jnp.float32),
        pltpu.VMEM_SHARED((CNT_WORDS,), jnp.float32),
        pltpu.SemaphoreType.DMA,
        pltpu.SemaphoreType.DMA,
    ],
)
def _eb_hist_kernel(idx_hbm, w00, w01, w02, w10, w11, w12, w20, w21, w22,
                    out_hbm,
                    idx_bufs, ones_v, cnt_v, wa0, wa1, wa2, wb0, wb1, wb2,
                    out_v, counts_sp, sem_a, sem_b):
    cid = lax.axis_index("c")
    sid = lax.axis_index("s")
    wid = sid * NUM_CORES + cid

    zero16 = jnp.zeros((L,), jnp.float32)
    one16 = zero16 + 1.0

    def fill_ones(i, _, ones_v=ones_v):
        ones_v[pl.ds(i * L, L)] = one16
        return 0

    lax.fori_loop(0, IC // L, fill_ones, 0)

    def fill_zero(i, _, cnt_v=cnt_v):
        cnt_v[pl.ds(i * L, L)] = zero16
        return 0

    lax.fori_loop(0, VC // L, fill_zero, 0)

    def local_start(it, sid=sid):
        return pl.multiple_of((sid + NUM_SUBCORES * it) * VC, 8)

    n_my_chunks = jnp.where(sid < EXTRA, CPT + 1, CPT)

    def zero_chunk(it, _, cnt_v=cnt_v, counts_sp=counts_sp):
        pltpu.sync_copy(cnt_v, counts_sp.at[pl.ds(local_start(it), VC)])
        return 0

    lax.fori_loop(0, n_my_chunks, zero_chunk, 0)
    plsc.subcore_barrier()

    lo = cid * HALF
    for j, buf in enumerate(idx_bufs):
        pltpu.sync_copy(idx_hbm.at[sid, j], buf)

    for buf in idx_bufs:
        def remap(i, _, buf=buf):
            s = pl.ds(i * L, L)
            rel = buf[s] - lo
            ok = (rel >= 0) & (rel < HALF)
            buf[s] = jnp.where(ok, rel, DUMMY)
            return 0

        lax.fori_loop(0, IC // L, remap, 0)
    plsc.subcore_barrier()

    for turn in range(NUM_SUBCORES):
        @pl.when(sid == turn)
        def _turn():
            for buf in idx_bufs:
                pltpu.sync_copy(ones_v, counts_sp.at[buf], add=True)
        plsc.subcore_barrier()

    tables = (((w00, w01, w02), SCALES[0]),
              ((w10, w11, w12), SCALES[1]),
              ((w20, w21, w22), SCALES[2]))
    slots = ((wa0, wa1, wa2), (wb0, wb1, wb2))
    sems = (sem_a, sem_b)

    def fire_table(g, it, slot, lo=lo):
        planes = tables[g][0]
        start = lo + local_start(it)
        return [pltpu.async_copy(planes[c].at[pl.ds(start, VC)],
                                 slots[slot][c], sems[slot])
                for c in range(EMBED_DIM)]

    def reduce_table(accs, g, slot, cnt_v=cnt_v):
        w0_v, w1_v, w2_v = slots[slot]

        def body(i, carry, w0_v=w0_v, w1_v=w1_v, w2_v=w2_v):
            a0, a1, a2 = carry
            s = pl.ds(i * L, L)
            cnt16 = cnt_v[s]
            a0 = a0 + cnt16 * w0_v[s]
            a1 = a1 + cnt16 * w1_v[s]
            a2 = a2 + cnt16 * w2_v[s]
            return a0, a1, a2

        new = lax.fori_loop(0, VC // L, body,
                            (accs[3 * g], accs[3 * g + 1], accs[3 * g + 2]))
        accs = list(accs)
        accs[3 * g], accs[3 * g + 1], accs[3 * g + 2] = new
        return tuple(accs)

    def chunk_body(it, accs):
        pltpu.sync_copy(counts_sp.at[pl.ds(local_start(it), VC)], cnt_v)
        descs = fire_table(0, it, 0)
        for g in range(3):
            nxt = fire_table(g + 1, it, (g + 1) % 2) if g < 2 else None
            for d in descs:
                d.wait()
            accs = reduce_table(accs, g, g % 2)
            descs = nxt
        return accs

    accs = lax.fori_loop(0, n_my_chunks, chunk_body,
                         tuple(zero16 for _ in range(9)))

    for p in range(9):
        out_v[p, :] = accs[p] * tables[p // 3][1]

    pltpu.sync_copy(out_v, out_hbm.at[wid])


def kernel(eb_input, eb_offset, W0, W1, W2):
    del eb_offset
    idx = eb_input.astype(jnp.int32).reshape(NUM_SUBCORES, NIC, IC)

    planes = [W[:, c] for W in (W0, W1, W2) for c in range(EMBED_DIM)]
    partials = _eb_hist_kernel(idx, *planes)
    return jnp.sum(partials, axis=(0, 2)).reshape(3, EMBED_DIM)


# speedup vs baseline: 331.6402x; 1.9848x over previous
import functools

import jax
import jax.numpy as jnp
from jax import lax
from jax.experimental import pallas as pl
from jax.experimental.pallas import tpu as pltpu
from jax.experimental.pallas import tpu_sc as plsc

NUM_EMBEDDINGS = 1_000_000
N_INDICES = 819200
EMBED_DIM = 3
NUM_CORES = 2
NUM_SUBCORES = 16
L = 16
NW = NUM_CORES * NUM_SUBCORES
B = N_INDICES // NW
IC = 6400
NIC = B // IC
SCALES = (5.0, 10.0, 6.0)

CNT_WORDS = NUM_EMBEDDINGS

VC = 4000
NCH = NUM_EMBEDDINGS // VC
CPT = NCH // NUM_SUBCORES
EXTRA = NCH - CPT * NUM_SUBCORES

_mesh = plsc.VectorSubcoreMesh(
    core_axis_name="c", subcore_axis_name="s",
    num_cores=NUM_CORES, num_subcores=NUM_SUBCORES)


@functools.partial(
    pl.kernel,
    out_type=jax.ShapeDtypeStruct((NW, 9, L), jnp.float32),
    mesh=_mesh,
    scratch_types=[
        [pltpu.VMEM((IC,), jnp.int32) for _ in range(NIC)],
        pltpu.VMEM((IC,), jnp.float32),
        pltpu.VMEM((VC,), jnp.float32),
        pltpu.VMEM((VC,), jnp.float32),
        pltpu.VMEM((VC,), jnp.float32),
        pltpu.VMEM((VC,), jnp.float32),
        pltpu.VMEM((VC,), jnp.float32),
        pltpu.VMEM((VC,), jnp.float32),
        pltpu.VMEM((VC,), jnp.float32),
        pltpu.VMEM((9, L), jnp.float32),
        pltpu.VMEM_SHARED((CNT_WORDS,), jnp.float32),
        pltpu.SemaphoreType.DMA,
        pltpu.SemaphoreType.DMA,
    ],
)
def _eb_hist_kernel(idx_hbm, w00, w01, w02, w10, w11, w12, w20, w21, w22,
                    out_hbm,
                    idx_bufs, ones_v, cnt_v, wa0, wa1, wa2, wb0, wb1, wb2,
                    out_v, counts_sp, sem_a, sem_b):
    cid = lax.axis_index("c")
    sid = lax.axis_index("s")
    wid = sid * NUM_CORES + cid

    zero16 = jnp.zeros((L,), jnp.float32)
    one16 = zero16 + 1.0

    def fill_ones(i, _, ones_v=ones_v):
        ones_v[pl.ds(i * L, L)] = one16
        return 0

    lax.fori_loop(0, IC // L, fill_ones, 0)

    def fill_zero(i, _, cnt_v=cnt_v):
        cnt_v[pl.ds(i * L, L)] = zero16
        return 0

    lax.fori_loop(0, VC // L, fill_zero, 0)

    def local_start(it, sid=sid):
        return pl.multiple_of((sid + NUM_SUBCORES * it) * VC, 8)

    n_my_chunks = jnp.where(sid < EXTRA, CPT + 1, CPT)

    def zero_chunk(it, _, cnt_v=cnt_v, counts_sp=counts_sp):
        pltpu.sync_copy(cnt_v, counts_sp.at[pl.ds(local_start(it), VC)])
        return 0

    lax.fori_loop(0, n_my_chunks, zero_chunk, 0)
    plsc.subcore_barrier()

    for j, buf in enumerate(idx_bufs):
        pltpu.sync_copy(idx_hbm.at[wid, j], buf)
    plsc.subcore_barrier()

    for turn in range(NUM_SUBCORES):
        @pl.when(sid == turn)
        def _turn():
            for buf in idx_bufs:
                pltpu.sync_copy(ones_v, counts_sp.at[buf], add=True)
        plsc.subcore_barrier()

    tables = (((w00, w01, w02), SCALES[0]),
              ((w10, w11, w12), SCALES[1]),
              ((w20, w21, w22), SCALES[2]))
    slots = ((wa0, wa1, wa2), (wb0, wb1, wb2))
    sems = (sem_a, sem_b)

    def fire_table(g, it, slot):
        planes = tables[g][0]
        start = local_start(it)
        return [pltpu.async_copy(planes[c].at[pl.ds(start, VC)],
                                 slots[slot][c], sems[slot])
                for c in range(EMBED_DIM)]

    def reduce_table(accs, g, slot, cnt_v=cnt_v):
        w0_v, w1_v, w2_v = slots[slot]

        def body(i, carry, w0_v=w0_v, w1_v=w1_v, w2_v=w2_v):
            a0, a1, a2 = carry
            for u in range(2):
                s = pl.ds((2 * i + u) * L, L)
                cnt16 = cnt_v[s]
                a0 = a0 + cnt16 * w0_v[s]
                a1 = a1 + cnt16 * w1_v[s]
                a2 = a2 + cnt16 * w2_v[s]
            return a0, a1, a2

        new = lax.fori_loop(0, VC // (2 * L), body,
                            (accs[3 * g], accs[3 * g + 1], accs[3 * g + 2]))
        accs = list(accs)
        accs[3 * g], accs[3 * g + 1], accs[3 * g + 2] = new
        return tuple(accs)

    def chunk_body(it, accs):
        pltpu.sync_copy(counts_sp.at[pl.ds(local_start(it), VC)], cnt_v)
        descs = fire_table(0, it, 0)
        for g in range(3):
            nxt = fire_table(g + 1, it, (g + 1) % 2) if g < 2 else None
            for d in descs:
                d.wait()
            accs = reduce_table(accs, g, g % 2)
            descs = nxt
        return accs

    accs = lax.fori_loop(0, n_my_chunks, chunk_body,
                         tuple(zero16 for _ in range(9)))

    for p in range(9):
        out_v[p, :] = accs[p] * tables[p // 3][1]

    pltpu.sync_copy(out_v, out_hbm.at[wid])


def kernel(eb_input, eb_offset, W0, W1, W2):
    del eb_offset
    idx = eb_input.astype(jnp.int32).reshape(NW, NIC, IC)

    planes = [W[:, c] for W in (W0, W1, W2) for c in range(EMBED_DIM)]
    partials = _eb_hist_kernel(idx, *planes)
    return jnp.sum(partials, axis=(0, 2)).reshape(3, EMBED_DIM)
